# baseline (speedup 1.0000x reference)
import math

import jax
import jax.numpy as jnp
from jax import lax
from jax.experimental import pallas as pl
from jax.experimental.pallas import tpu as pltpu
from jax.experimental.pallas import tpu_sc as plsc

_Q = 2048
_NHEAD = 16
_NBUCKET = 32
_HS = 4
_LPAD = 4224
_LROW = 4096
_LROW2 = 3968
_RI = 256


def _line_t(bucket_ref, embT_ref):
    bucket = jnp.broadcast_to(bucket_ref[...], (_NBUCKET, _LPAD))
    ids = lax.broadcasted_iota(jnp.int32, (_NBUCKET, _LPAD), 0)
    oh = (bucket == ids).astype(jnp.float32)
    return jnp.dot(embT_ref[...], oh, preferred_element_type=jnp.float32,
                   precision=lax.Precision.HIGHEST)


def _tc_linel_body(bucket_ref, embT_ref, lineL_ref):
    lineT = _line_t(bucket_ref, embT_ref)
    for r in range(8):
        lineL_ref[:, r, :] = lax.slice(lineT, (0, 7 - r), (_NHEAD, 7 - r + _LROW))


def _tc_linel(bucket, embT):
    return pl.pallas_call(
        _tc_linel_body,
        out_shape=jax.ShapeDtypeStruct((_NHEAD, 8, _LROW), jnp.float32),
    )(bucket, embT)


def _tc_l16_body(bucket_ref, embT_ref, l16_ref):
    bucket = jnp.broadcast_to(bucket_ref[0], (_NBUCKET, _LROW2 + 16))
    ids = lax.broadcasted_iota(jnp.int32, (_NBUCKET, _LROW2 + 16), 0)
    oh = (bucket == ids).astype(jnp.float32)
    lineV = jnp.dot(embT_ref[...], oh, preferred_element_type=jnp.float32,
                    precision=lax.Precision.HIGHEST)
    for r in range(8):
        l16_ref[:, 0, r, :] = lax.slice(lineV, (0, 7 - r), (_NHEAD, 7 - r + _LROW2))


def _tc_l16(bucket16, embT):
    return pl.pallas_call(
        _tc_l16_body,
        grid=(16,),
        in_specs=[
            pl.BlockSpec((1, 1, _LROW2 + 16), lambda v: (v, 0, 0)),
            pl.BlockSpec((_NHEAD, _NBUCKET), lambda v: (0, 0)),
        ],
        out_specs=pl.BlockSpec((_NHEAD, 1, 8, _LROW2), lambda v: (0, v, 0, 0)),
        out_shape=jax.ShapeDtypeStruct((_NHEAD, 16, 8, _LROW2), jnp.float32),
    )(bucket16, embT)


def _sc_body(lineL_hbm, out_hbm, lines_v, sem):
    nc = 2
    wid = lax.axis_index("s") * nc + lax.axis_index("c")
    wph = 32 // _HS
    h = wid // wph
    part = wid % wph
    rows = _Q // wph

    lsz = 8 * _LROW
    pltpu.sync_copy(lineL_hbm.at[pl.ds(pl.multiple_of(h * lsz, 8), lsz)], lines_v)

    base_i = part * rows

    def group8(g, carry):
        i0 = base_i + g * 8
        base = 2040 - i0
        descs = []
        for k in range(8):
            src = pl.multiple_of(k * _LROW + base, 8)
            d = pltpu.make_async_copy(
                lines_v.at[pl.ds(src, _Q)],
                out_hbm.at[pl.ds(pl.multiple_of((h * _Q + i0 + k) * _Q, 8), _Q)],
                sem,
            )
            d.start()
            descs.append(d)
        for d in descs:
            d.wait()
        return carry

    lax.fori_loop(0, rows // 8, group8, 0)


def _sc_call(lineL):
    mesh = plsc.VectorSubcoreMesh(core_axis_name="c", subcore_axis_name="s")
    return pl.kernel(
        _sc_body,
        out_type=jax.ShapeDtypeStruct((_HS * _Q * _Q,), jnp.float32),
        mesh=mesh,
        scratch_types=[
            pltpu.VMEM((8 * _LROW,), jnp.float32),
            pltpu.SemaphoreType.DMA,
        ],
    )(lineL)


def _tc_direct_body(l16_ref, out_ref):
    k = pl.program_id(1)
    off1 = pl.multiple_of(128 * (15 - 2 * k), 128)
    off2 = pl.multiple_of(128 * (14 - 2 * k), 128)
    for q in range(16):
        out_ref[0, 0, 8 * q:8 * q + 8, :] = l16_ref[0, 15 - q, :, pl.ds(off1, _Q)]
    for q in range(16, 32):
        out_ref[0, 0, 8 * q:8 * q + 8, :] = l16_ref[0, 31 - q, :, pl.ds(off2, _Q)]


def _tc_direct(l16):
    return pl.pallas_call(
        _tc_direct_body,
        grid=(_NHEAD - _HS, _Q // _RI),
        in_specs=[pl.BlockSpec(
            (1, 16, 8, _LROW2), lambda h, k: (_HS + h, 0, 0, 0))],
        out_specs=pl.BlockSpec(
            (1, 1, _RI, _Q), lambda h, k: (0, _HS + h, k, 0)),
        out_shape=jax.ShapeDtypeStruct((1, _NHEAD, _Q, _Q), jnp.float32),
    )(l16)


def _tc_finish_body(o1_ref, lin_ref, out_ref):
    out_ref[0, 0] = lin_ref[...].reshape(_RI, _Q)


def _tc_finish(o1, lin3):
    return pl.pallas_call(
        _tc_finish_body,
        grid=(_HS, _Q // _RI),
        in_specs=[
            pl.BlockSpec(memory_space=pl.ANY),
            pl.BlockSpec((_RI, 16, 128),
                         lambda h, k: (h * (_Q // _RI) + k, 0, 0)),
        ],
        out_specs=pl.BlockSpec(
            (1, 1, _RI, _Q), lambda h, k: (0, h, k, 0)),
        out_shape=jax.ShapeDtypeStruct((1, _NHEAD, _Q, _Q), jnp.float32),
        input_output_aliases={0: 0},
    )(o1, lin3)


def kernel(q_len, k_len, bias_emb):
    if k_len is None:
        k_len = q_len
    t = jnp.arange(_LPAD, dtype=jnp.int32)
    d = t - 2047
    rb = jnp.where(d > 0, 16, 0).astype(jnp.int32)
    a = jnp.abs(d)
    rp_safe = jnp.maximum(a, 1)
    large = 8 + (
        jnp.log(rp_safe.astype(jnp.float32) / 8) / math.log(16.0) * 8
    ).astype(jnp.int32)
    large = jnp.minimum(large, 15)
    bucket = (rb + jnp.where(a < 8, a, large)).reshape(1, _LPAD)
    embT = bias_emb.T
    idx = (jnp.arange(_LROW2 + 16, dtype=jnp.int32)[None, :]
           + 8 * jnp.arange(16, dtype=jnp.int32)[:, None])
    bucket16 = jnp.take(bucket[0], idx, axis=0).reshape(16, 1, _LROW2 + 16)
    lineL = _tc_linel(bucket, embT)
    lin = _sc_call(lineL.reshape(_NHEAD * 8 * _LROW))
    l16 = _tc_l16(bucket16, embT)
    o1 = _tc_direct(l16)
    lin3 = lin.reshape(_HS * _Q, 16, 128)
    return _tc_finish(o1, lin3)


# speedup vs baseline: 2.5811x; 2.5811x over previous
import math

import jax
import jax.numpy as jnp
from jax import lax
from jax.experimental import pallas as pl
from jax.experimental.pallas import tpu as pltpu
from jax.experimental.pallas import tpu_sc as plsc

_Q = 2048
_NHEAD = 16
_NBUCKET = 32
_HS = 4
_LPAD = 4224
_LROW = 4096
_LROW2 = 3968
_RI = 256


def _line_t(bucket_ref, embT_ref):
    bucket = jnp.broadcast_to(bucket_ref[...], (_NBUCKET, _LPAD))
    ids = lax.broadcasted_iota(jnp.int32, (_NBUCKET, _LPAD), 0)
    oh = (bucket == ids).astype(jnp.float32)
    return jnp.dot(embT_ref[...], oh, preferred_element_type=jnp.float32,
                   precision=lax.Precision.HIGHEST)


def _tc_linel_body(bucket_ref, embT_ref, lineL_ref):
    lineT = _line_t(bucket_ref, embT_ref)
    for r in range(8):
        lineL_ref[:, r, :] = lax.slice(lineT, (0, 7 - r), (_NHEAD, 7 - r + _LROW))


def _tc_linel(bucket, embT):
    return pl.pallas_call(
        _tc_linel_body,
        out_shape=jax.ShapeDtypeStruct((_NHEAD, 8, _LROW), jnp.float32),
    )(bucket, embT)


def _tc_l16_body(bucket_ref, embT_ref, l16_ref):
    lineT = _line_t(bucket_ref, embT_ref)
    for v in range(16):
        for r in range(8):
            s = 8 * v + 7 - r
            l16_ref[:, v, r, :] = lax.slice(lineT, (0, s), (_NHEAD, s + _LROW2))


def _tc_l16(bucket, embT):
    return pl.pallas_call(
        _tc_l16_body,
        out_shape=jax.ShapeDtypeStruct((_NHEAD, 16, 8, _LROW2), jnp.float32),
    )(bucket, embT)


def _sc_body(lineL_hbm, out_hbm, lines_v, sem):
    nc = 2
    wid = lax.axis_index("s") * nc + lax.axis_index("c")
    wph = 32 // _HS
    h = wid // wph
    part = wid % wph
    rows = _Q // wph

    lsz = 8 * _LROW
    pltpu.sync_copy(lineL_hbm.at[pl.ds(pl.multiple_of(h * lsz, 8), lsz)], lines_v)

    base_i = part * rows

    def group8(g, carry):
        i0 = base_i + g * 8
        base = 2040 - i0
        descs = []
        for k in range(8):
            src = pl.multiple_of(k * _LROW + base, 8)
            d = pltpu.make_async_copy(
                lines_v.at[pl.ds(src, _Q)],
                out_hbm.at[pl.ds(pl.multiple_of((h * _Q + i0 + k) * _Q, 8), _Q)],
                sem,
            )
            d.start()
            descs.append(d)
        for d in descs:
            d.wait()
        return carry

    lax.fori_loop(0, rows // 8, group8, 0)


def _sc_call(lineL):
    mesh = plsc.VectorSubcoreMesh(core_axis_name="c", subcore_axis_name="s")
    return pl.kernel(
        _sc_body,
        out_type=jax.ShapeDtypeStruct((_HS * _Q * _Q,), jnp.float32),
        mesh=mesh,
        scratch_types=[
            pltpu.VMEM((8 * _LROW,), jnp.float32),
            pltpu.SemaphoreType.DMA,
        ],
    )(lineL)


def _tc_direct_body(l16_ref, out_ref):
    k = pl.program_id(1)
    off1 = pl.multiple_of(128 * (15 - 2 * k), 128)
    off2 = pl.multiple_of(128 * (14 - 2 * k), 128)
    for q in range(16):
        out_ref[0, 0, 8 * q:8 * q + 8, :] = l16_ref[0, 15 - q, :, pl.ds(off1, _Q)]
    for q in range(16, 32):
        out_ref[0, 0, 8 * q:8 * q + 8, :] = l16_ref[0, 31 - q, :, pl.ds(off2, _Q)]


def _tc_direct(l16):
    return pl.pallas_call(
        _tc_direct_body,
        grid=(_NHEAD - _HS, _Q // _RI),
        in_specs=[pl.BlockSpec(
            (1, 16, 8, _LROW2), lambda h, k: (_HS + h, 0, 0, 0))],
        out_specs=pl.BlockSpec(
            (1, 1, _RI, _Q), lambda h, k: (0, _HS + h, k, 0)),
        out_shape=jax.ShapeDtypeStruct((1, _NHEAD, _Q, _Q), jnp.float32),
    )(l16)


def _tc_finish_body(o1_ref, lin_ref, out_ref):
    out_ref[0, 0] = lin_ref[...].reshape(_RI, _Q)


def _tc_finish(o1, lin3):
    return pl.pallas_call(
        _tc_finish_body,
        grid=(_HS, _Q // _RI),
        in_specs=[
            pl.BlockSpec(memory_space=pl.ANY),
            pl.BlockSpec((_RI, 16, 128),
                         lambda h, k: (h * (_Q // _RI) + k, 0, 0)),
        ],
        out_specs=pl.BlockSpec(
            (1, 1, _RI, _Q), lambda h, k: (0, h, k, 0)),
        out_shape=jax.ShapeDtypeStruct((1, _NHEAD, _Q, _Q), jnp.float32),
        input_output_aliases={0: 0},
    )(o1, lin3)


def kernel(q_len, k_len, bias_emb):
    if k_len is None:
        k_len = q_len
    t = jnp.arange(_LPAD, dtype=jnp.int32)
    d = t - 2047
    rb = jnp.where(d > 0, 16, 0).astype(jnp.int32)
    a = jnp.abs(d)
    rp_safe = jnp.maximum(a, 1)
    large = 8 + (
        jnp.log(rp_safe.astype(jnp.float32) / 8) / math.log(16.0) * 8
    ).astype(jnp.int32)
    large = jnp.minimum(large, 15)
    bucket = (rb + jnp.where(a < 8, a, large)).reshape(1, _LPAD)
    embT = bias_emb.T
    lineL = _tc_linel(bucket, embT)
    lin = _sc_call(lineL.reshape(_NHEAD * 8 * _LROW))
    l16 = _tc_l16(bucket, embT)
    o1 = _tc_direct(l16)
    lin3 = lin.reshape(_HS * _Q, 16, 128)
    return _tc_finish(o1, lin3)


# speedup vs baseline: 2.7884x; 1.0803x over previous
import math

import jax
import jax.numpy as jnp
from jax import lax
from jax.experimental import pallas as pl
from jax.experimental.pallas import tpu as pltpu
from jax.experimental.pallas import tpu_sc as plsc

_Q = 2048
_NHEAD = 16
_NBUCKET = 32
_HS = 4
_LPAD = 4224
_LROW = 4096
_LROW2 = 3968
_RI = 512


def _line_t(bucket_ref, embT_ref):
    bucket = jnp.broadcast_to(bucket_ref[...], (_NBUCKET, _LPAD))
    ids = lax.broadcasted_iota(jnp.int32, (_NBUCKET, _LPAD), 0)
    oh = (bucket == ids).astype(jnp.float32)
    return jnp.dot(embT_ref[...], oh, preferred_element_type=jnp.float32,
                   precision=lax.Precision.HIGHEST)


def _tc_linel_body(bucket_ref, embT_ref, lineL_ref):
    lineT = _line_t(bucket_ref, embT_ref)
    for r in range(8):
        lineL_ref[:, r, :] = lax.slice(lineT, (0, 7 - r), (_NHEAD, 7 - r + _LROW))


def _tc_linel(bucket, embT):
    return pl.pallas_call(
        _tc_linel_body,
        out_shape=jax.ShapeDtypeStruct((_NHEAD, 8, _LROW), jnp.float32),
    )(bucket, embT)


def _tc_l16_body(bucket_ref, embT_ref, l16_ref):
    lineT = _line_t(bucket_ref, embT_ref)
    for v in range(16):
        for r in range(8):
            s = 8 * v + 7 - r
            l16_ref[:, v, r, :] = lax.slice(lineT, (0, s), (_NHEAD, s + _LROW2))


def _tc_l16(bucket, embT):
    return pl.pallas_call(
        _tc_l16_body,
        out_shape=jax.ShapeDtypeStruct((_NHEAD, 16, 8, _LROW2), jnp.float32),
    )(bucket, embT)


def _sc_body(lineL_hbm, out_hbm, lines_v, sem):
    nc = 2
    wid = lax.axis_index("s") * nc + lax.axis_index("c")
    wph = 32 // _HS
    h = wid // wph
    part = wid % wph
    rows = _Q // wph

    lsz = 8 * _LROW
    pltpu.sync_copy(lineL_hbm.at[pl.ds(pl.multiple_of(h * lsz, 8), lsz)], lines_v)

    base_i = part * rows

    def group8(g, carry):
        i0 = base_i + g * 8
        base = 2040 - i0
        descs = []
        for k in range(8):
            src = pl.multiple_of(k * _LROW + base, 8)
            d = pltpu.make_async_copy(
                lines_v.at[pl.ds(src, _Q)],
                out_hbm.at[pl.ds(pl.multiple_of((h * _Q + i0 + k) * _Q, 8), _Q)],
                sem,
            )
            d.start()
            descs.append(d)
        for d in descs:
            d.wait()
        return carry

    lax.fori_loop(0, rows // 8, group8, 0)


def _sc_call(lineL):
    mesh = plsc.VectorSubcoreMesh(core_axis_name="c", subcore_axis_name="s")
    return pl.kernel(
        _sc_body,
        out_type=jax.ShapeDtypeStruct((_HS * _Q * _Q,), jnp.float32),
        mesh=mesh,
        scratch_types=[
            pltpu.VMEM((8 * _LROW,), jnp.float32),
            pltpu.SemaphoreType.DMA,
        ],
    )(lineL)


def _tc_direct_body(l16_ref, out_ref):
    k = pl.program_id(1)
    for j in range(_RI // 128):
        off = pl.multiple_of(128 * (15 - (_RI // 128) * k - j), 128)
        for qq in range(16):
            q = 16 * j + qq
            out_ref[0, 0, 8 * q:8 * q + 8, :] = \
                l16_ref[0, 15 - qq, :, pl.ds(off, _Q)]


def _tc_direct(l16):
    return pl.pallas_call(
        _tc_direct_body,
        grid=(_NHEAD - _HS, _Q // _RI),
        in_specs=[pl.BlockSpec(
            (1, 16, 8, _LROW2), lambda h, k: (_HS + h, 0, 0, 0))],
        out_specs=pl.BlockSpec(
            (1, 1, _RI, _Q), lambda h, k: (0, _HS + h, k, 0)),
        out_shape=jax.ShapeDtypeStruct((1, _NHEAD, _Q, _Q), jnp.float32),
    )(l16)


def _tc_finish_body(o1_ref, lin_ref, out_ref):
    out_ref[0, 0] = lin_ref[...].reshape(_RI, _Q)


def _tc_finish(o1, lin3):
    return pl.pallas_call(
        _tc_finish_body,
        grid=(_HS, _Q // _RI),
        in_specs=[
            pl.BlockSpec(memory_space=pl.ANY),
            pl.BlockSpec((_RI, 16, 128),
                         lambda h, k: (h * (_Q // _RI) + k, 0, 0)),
        ],
        out_specs=pl.BlockSpec(
            (1, 1, _RI, _Q), lambda h, k: (0, h, k, 0)),
        out_shape=jax.ShapeDtypeStruct((1, _NHEAD, _Q, _Q), jnp.float32),
        input_output_aliases={0: 0},
    )(o1, lin3)


def kernel(q_len, k_len, bias_emb):
    if k_len is None:
        k_len = q_len
    t = jnp.arange(_LPAD, dtype=jnp.int32)
    d = t - 2047
    rb = jnp.where(d > 0, 16, 0).astype(jnp.int32)
    a = jnp.abs(d)
    rp_safe = jnp.maximum(a, 1)
    large = 8 + (
        jnp.log(rp_safe.astype(jnp.float32) / 8) / math.log(16.0) * 8
    ).astype(jnp.int32)
    large = jnp.minimum(large, 15)
    bucket = (rb + jnp.where(a < 8, a, large)).reshape(1, _LPAD)
    embT = bias_emb.T
    lineL = _tc_linel(bucket, embT)
    lin = _sc_call(lineL.reshape(_NHEAD * 8 * _LROW))
    l16 = _tc_l16(bucket, embT)
    o1 = _tc_direct(l16)
    lin3 = lin.reshape(_HS * _Q, 16, 128)
    return _tc_finish(o1, lin3)


# speedup vs baseline: 3.0226x; 1.0840x over previous
import math

import jax
import jax.numpy as jnp
from jax import lax
from jax.experimental import pallas as pl
from jax.experimental.pallas import tpu as pltpu
from jax.experimental.pallas import tpu_sc as plsc

_Q = 2048
_NHEAD = 16
_NBUCKET = 32
_HS = 4
_LPAD = 4224
_LROW = 4096
_LROW2 = 3968
_RI = 1024


def _line_t(bucket_ref, embT_ref):
    bucket = jnp.broadcast_to(bucket_ref[...], (_NBUCKET, _LPAD))
    ids = lax.broadcasted_iota(jnp.int32, (_NBUCKET, _LPAD), 0)
    oh = (bucket == ids).astype(jnp.float32)
    return jnp.dot(embT_ref[...], oh, preferred_element_type=jnp.float32,
                   precision=lax.Precision.HIGHEST)


def _tc_linel_body(bucket_ref, embT_ref, lineL_ref):
    lineT = _line_t(bucket_ref, embT_ref)
    for r in range(8):
        lineL_ref[:, r, :] = lax.slice(lineT, (0, 7 - r), (_NHEAD, 7 - r + _LROW))


def _tc_linel(bucket, embT):
    return pl.pallas_call(
        _tc_linel_body,
        out_shape=jax.ShapeDtypeStruct((_NHEAD, 8, _LROW), jnp.float32),
    )(bucket, embT)


def _tc_l16_body(bucket_ref, embT_ref, l16_ref):
    lineT = _line_t(bucket_ref, embT_ref)
    for v in range(16):
        for r in range(8):
            s = 8 * v + 7 - r
            l16_ref[:, v, r, :] = lax.slice(lineT, (0, s), (_NHEAD, s + _LROW2))


def _tc_l16(bucket, embT):
    return pl.pallas_call(
        _tc_l16_body,
        out_shape=jax.ShapeDtypeStruct((_NHEAD, 16, 8, _LROW2), jnp.float32),
    )(bucket, embT)


def _sc_body(lineL_hbm, out_hbm, lines_v, sem):
    nc = 2
    wid = lax.axis_index("s") * nc + lax.axis_index("c")
    wph = 32 // _HS
    h = wid // wph
    part = wid % wph
    rows = _Q // wph

    lsz = 8 * _LROW
    pltpu.sync_copy(lineL_hbm.at[pl.ds(pl.multiple_of(h * lsz, 8), lsz)], lines_v)

    base_i = part * rows

    def group8(g, carry):
        i0 = base_i + g * 8
        base = 2040 - i0
        descs = []
        for k in range(8):
            src = pl.multiple_of(k * _LROW + base, 8)
            d = pltpu.make_async_copy(
                lines_v.at[pl.ds(src, _Q)],
                out_hbm.at[pl.ds(pl.multiple_of((h * _Q + i0 + k) * _Q, 8), _Q)],
                sem,
            )
            d.start()
            descs.append(d)
        for d in descs:
            d.wait()
        return carry

    lax.fori_loop(0, rows // 8, group8, 0)


def _sc_call(lineL):
    mesh = plsc.VectorSubcoreMesh(core_axis_name="c", subcore_axis_name="s")
    return pl.kernel(
        _sc_body,
        out_type=jax.ShapeDtypeStruct((_HS * _Q * _Q,), jnp.float32),
        mesh=mesh,
        scratch_types=[
            pltpu.VMEM((8 * _LROW,), jnp.float32),
            pltpu.SemaphoreType.DMA,
        ],
    )(lineL)


def _tc_direct_body(l16_ref, out_ref):
    k = pl.program_id(1)
    for j in range(_RI // 128):
        off = pl.multiple_of(128 * (15 - (_RI // 128) * k - j), 128)
        for qq in range(16):
            q = 16 * j + qq
            out_ref[0, 0, 8 * q:8 * q + 8, :] = \
                l16_ref[0, 15 - qq, :, pl.ds(off, _Q)]


def _tc_direct(l16):
    return pl.pallas_call(
        _tc_direct_body,
        grid=(_NHEAD - _HS, _Q // _RI),
        in_specs=[pl.BlockSpec(
            (1, 16, 8, _LROW2), lambda h, k: (_HS + h, 0, 0, 0))],
        out_specs=pl.BlockSpec(
            (1, 1, _RI, _Q), lambda h, k: (0, _HS + h, k, 0)),
        out_shape=jax.ShapeDtypeStruct((1, _NHEAD, _Q, _Q), jnp.float32),
    )(l16)


def _tc_finish_body(o1_ref, lin_ref, out_ref):
    out_ref[0, 0] = lin_ref[...].reshape(_RI, _Q)


def _tc_finish(o1, lin3):
    return pl.pallas_call(
        _tc_finish_body,
        grid=(_HS, _Q // _RI),
        in_specs=[
            pl.BlockSpec(memory_space=pl.ANY),
            pl.BlockSpec((_RI, 16, 128),
                         lambda h, k: (h * (_Q // _RI) + k, 0, 0)),
        ],
        out_specs=pl.BlockSpec(
            (1, 1, _RI, _Q), lambda h, k: (0, h, k, 0)),
        out_shape=jax.ShapeDtypeStruct((1, _NHEAD, _Q, _Q), jnp.float32),
        input_output_aliases={0: 0},
    )(o1, lin3)


def kernel(q_len, k_len, bias_emb):
    if k_len is None:
        k_len = q_len
    t = jnp.arange(_LPAD, dtype=jnp.int32)
    d = t - 2047
    rb = jnp.where(d > 0, 16, 0).astype(jnp.int32)
    a = jnp.abs(d)
    rp_safe = jnp.maximum(a, 1)
    large = 8 + (
        jnp.log(rp_safe.astype(jnp.float32) / 8) / math.log(16.0) * 8
    ).astype(jnp.int32)
    large = jnp.minimum(large, 15)
    bucket = (rb + jnp.where(a < 8, a, large)).reshape(1, _LPAD)
    embT = bias_emb.T
    lineL = _tc_linel(bucket, embT)
    lin = _sc_call(lineL.reshape(_NHEAD * 8 * _LROW))
    l16 = _tc_l16(bucket, embT)
    o1 = _tc_direct(l16)
    lin3 = lin.reshape(_HS * _Q, 16, 128)
    return _tc_finish(o1, lin3)


# speedup vs baseline: 3.0278x; 1.0017x over previous
import math

import jax
import jax.numpy as jnp
from jax import lax
from jax.experimental import pallas as pl
from jax.experimental.pallas import tpu as pltpu
from jax.experimental.pallas import tpu_sc as plsc

_Q = 2048
_NHEAD = 16
_NBUCKET = 32
_HS = 4
_LPAD = 4224
_LROW = 4096
_LROW2 = 3968
_RI = 1024
_RID = 2048


def _line_t(bucket_ref, embT_ref):
    bucket = jnp.broadcast_to(bucket_ref[...], (_NBUCKET, _LPAD))
    ids = lax.broadcasted_iota(jnp.int32, (_NBUCKET, _LPAD), 0)
    oh = (bucket == ids).astype(jnp.float32)
    return jnp.dot(embT_ref[...], oh, preferred_element_type=jnp.float32,
                   precision=lax.Precision.HIGHEST)


def _tc_linel_body(bucket_ref, embT_ref, lineL_ref):
    lineT = _line_t(bucket_ref, embT_ref)
    for r in range(8):
        lineL_ref[:, r, :] = lax.slice(lineT, (0, 7 - r), (_NHEAD, 7 - r + _LROW))


def _tc_linel(bucket, embT):
    return pl.pallas_call(
        _tc_linel_body,
        out_shape=jax.ShapeDtypeStruct((_NHEAD, 8, _LROW), jnp.float32),
    )(bucket, embT)


def _tc_l16_body(bucket_ref, embT_ref, l16_ref):
    lineT = _line_t(bucket_ref, embT_ref)
    for v in range(16):
        for r in range(8):
            s = 8 * v + 7 - r
            l16_ref[:, v, r, :] = lax.slice(lineT, (0, s), (_NHEAD, s + _LROW2))


def _tc_l16(bucket, embT):
    return pl.pallas_call(
        _tc_l16_body,
        out_shape=jax.ShapeDtypeStruct((_NHEAD, 16, 8, _LROW2), jnp.float32),
    )(bucket, embT)


def _sc_body(lineL_hbm, out_hbm, lines_v, sem):
    nc = 2
    wid = lax.axis_index("s") * nc + lax.axis_index("c")
    wph = 32 // _HS
    h = wid // wph
    part = wid % wph
    rows = _Q // wph

    lsz = 8 * _LROW
    pltpu.sync_copy(lineL_hbm.at[pl.ds(pl.multiple_of(h * lsz, 8), lsz)], lines_v)

    base_i = part * rows

    def group8(g, carry):
        i0 = base_i + g * 8
        base = 2040 - i0
        descs = []
        for k in range(8):
            src = pl.multiple_of(k * _LROW + base, 8)
            d = pltpu.make_async_copy(
                lines_v.at[pl.ds(src, _Q)],
                out_hbm.at[pl.ds(pl.multiple_of((h * _Q + i0 + k) * _Q, 8), _Q)],
                sem,
            )
            d.start()
            descs.append(d)
        for d in descs:
            d.wait()
        return carry

    lax.fori_loop(0, rows // 8, group8, 0)


def _sc_call(lineL):
    mesh = plsc.VectorSubcoreMesh(core_axis_name="c", subcore_axis_name="s")
    return pl.kernel(
        _sc_body,
        out_type=jax.ShapeDtypeStruct((_HS * _Q * _Q,), jnp.float32),
        mesh=mesh,
        scratch_types=[
            pltpu.VMEM((8 * _LROW,), jnp.float32),
            pltpu.SemaphoreType.DMA,
        ],
    )(lineL)


def _tc_direct_body(l16_ref, out_ref):
    k = pl.program_id(1)
    for j in range(_RID // 128):
        off = pl.multiple_of(128 * (15 - (_RID // 128) * k - j), 128)
        for qq in range(16):
            q = 16 * j + qq
            out_ref[0, 0, 8 * q:8 * q + 8, :] = \
                l16_ref[0, 15 - qq, :, pl.ds(off, _Q)]


def _tc_direct(l16):
    return pl.pallas_call(
        _tc_direct_body,
        grid=(_NHEAD - _HS, _Q // _RID),
        in_specs=[pl.BlockSpec(
            (1, 16, 8, _LROW2), lambda h, k: (_HS + h, 0, 0, 0))],
        out_specs=pl.BlockSpec(
            (1, 1, _RID, _Q), lambda h, k: (0, _HS + h, k, 0)),
        out_shape=jax.ShapeDtypeStruct((1, _NHEAD, _Q, _Q), jnp.float32),
    )(l16)


def _tc_finish_body(o1_ref, lin_ref, out_ref):
    out_ref[0, 0] = lin_ref[...].reshape(_RI, _Q)


def _tc_finish(o1, lin3):
    return pl.pallas_call(
        _tc_finish_body,
        grid=(_HS, _Q // _RI),
        in_specs=[
            pl.BlockSpec(memory_space=pl.ANY),
            pl.BlockSpec((_RI, 16, 128),
                         lambda h, k: (h * (_Q // _RI) + k, 0, 0)),
        ],
        out_specs=pl.BlockSpec(
            (1, 1, _RI, _Q), lambda h, k: (0, h, k, 0)),
        out_shape=jax.ShapeDtypeStruct((1, _NHEAD, _Q, _Q), jnp.float32),
        input_output_aliases={0: 0},
    )(o1, lin3)


def kernel(q_len, k_len, bias_emb):
    if k_len is None:
        k_len = q_len
    t = jnp.arange(_LPAD, dtype=jnp.int32)
    d = t - 2047
    rb = jnp.where(d > 0, 16, 0).astype(jnp.int32)
    a = jnp.abs(d)
    rp_safe = jnp.maximum(a, 1)
    large = 8 + (
        jnp.log(rp_safe.astype(jnp.float32) / 8) / math.log(16.0) * 8
    ).astype(jnp.int32)
    large = jnp.minimum(large, 15)
    bucket = (rb + jnp.where(a < 8, a, large)).reshape(1, _LPAD)
    embT = bias_emb.T
    lineL = _tc_linel(bucket, embT)
    lin = _sc_call(lineL.reshape(_NHEAD * 8 * _LROW))
    l16 = _tc_l16(bucket, embT)
    o1 = _tc_direct(l16)
    lin3 = lin.reshape(_HS * _Q, 16, 128)
    return _tc_finish(o1, lin3)


# speedup vs baseline: 3.0865x; 1.0194x over previous
import math

import jax
import jax.numpy as jnp
from jax import lax
from jax.experimental import pallas as pl
from jax.experimental.pallas import tpu as pltpu
from jax.experimental.pallas import tpu_sc as plsc

_Q = 2048
_NHEAD = 16
_NBUCKET = 32
_HS = 4
_LPAD = 4224
_LROW = 4096
_LROW2 = 3968
_RI = 1024
_RID = 2048


def _line_t(bucket_ref, embT_ref):
    bucket = jnp.broadcast_to(bucket_ref[...], (_NBUCKET, _LPAD))
    ids = lax.broadcasted_iota(jnp.int32, (_NBUCKET, _LPAD), 0)
    oh = (bucket == ids).astype(jnp.float32)
    return jnp.dot(embT_ref[...], oh, preferred_element_type=jnp.float32,
                   precision=lax.Precision.HIGHEST)


def _tc_linel_body(bucket_ref, embT_ref, lineL_ref):
    lineT = _line_t(bucket_ref, embT_ref)
    for r in range(8):
        lineL_ref[:, r, :] = lax.slice(lineT, (0, 7 - r), (_NHEAD, 7 - r + _LROW))


def _tc_linel(bucket, embT):
    return pl.pallas_call(
        _tc_linel_body,
        out_shape=jax.ShapeDtypeStruct((_NHEAD, 8, _LROW), jnp.float32),
    )(bucket, embT)


def _tc_l16_body(bucket_ref, embT_ref, l16_ref):
    lineT = _line_t(bucket_ref, embT_ref)
    for v in range(16):
        for r in range(8):
            s = 8 * v + 7 - r
            l16_ref[:, v, r, :] = lax.slice(lineT, (_HS, s), (_NHEAD, s + _LROW2))


def _tc_l16(bucket, embT):
    return pl.pallas_call(
        _tc_l16_body,
        out_shape=jax.ShapeDtypeStruct((_NHEAD - _HS, 16, 8, _LROW2), jnp.float32),
    )(bucket, embT)


def _sc_body(lineL_hbm, out_hbm, lines_v, sem):
    nc = 2
    wid = lax.axis_index("s") * nc + lax.axis_index("c")
    wph = 32 // _HS
    h = wid // wph
    part = wid % wph
    rows = _Q // wph

    lsz = 8 * _LROW
    pltpu.sync_copy(lineL_hbm.at[pl.ds(pl.multiple_of(h * lsz, 8), lsz)], lines_v)

    base_i = part * rows

    def group8(g, carry):
        i0 = base_i + g * 8
        base = 2040 - i0
        descs = []
        for k in range(8):
            src = pl.multiple_of(k * _LROW + base, 8)
            d = pltpu.make_async_copy(
                lines_v.at[pl.ds(src, _Q)],
                out_hbm.at[pl.ds(pl.multiple_of((h * _Q + i0 + k) * _Q, 8), _Q)],
                sem,
            )
            d.start()
            descs.append(d)
        for d in descs:
            d.wait()
        return carry

    lax.fori_loop(0, rows // 8, group8, 0)


def _sc_call(lineL):
    mesh = plsc.VectorSubcoreMesh(core_axis_name="c", subcore_axis_name="s")
    return pl.kernel(
        _sc_body,
        out_type=jax.ShapeDtypeStruct((_HS * _Q * _Q,), jnp.float32),
        mesh=mesh,
        scratch_types=[
            pltpu.VMEM((8 * _LROW,), jnp.float32),
            pltpu.SemaphoreType.DMA,
        ],
    )(lineL)


def _tc_direct_body(l16_ref, out_ref):
    k = pl.program_id(1)
    for j in range(_RID // 128):
        off = pl.multiple_of(128 * (15 - (_RID // 128) * k - j), 128)
        for qq in range(16):
            q = 16 * j + qq
            out_ref[0, 0, 8 * q:8 * q + 8, :] = \
                l16_ref[0, 15 - qq, :, pl.ds(off, _Q)]


def _tc_direct(l16):
    return pl.pallas_call(
        _tc_direct_body,
        grid=(_NHEAD - _HS, _Q // _RID),
        in_specs=[pl.BlockSpec(
            (1, 16, 8, _LROW2), lambda h, k: (h, 0, 0, 0))],
        out_specs=pl.BlockSpec(
            (1, 1, _RID, _Q), lambda h, k: (0, _HS + h, k, 0)),
        out_shape=jax.ShapeDtypeStruct((1, _NHEAD, _Q, _Q), jnp.float32),
    )(l16)


def _tc_finish_body(o1_ref, lin_ref, out_ref):
    out_ref[0, 0] = lin_ref[...].reshape(_RI, _Q)


def _tc_finish(o1, lin3):
    return pl.pallas_call(
        _tc_finish_body,
        grid=(_HS, _Q // _RI),
        in_specs=[
            pl.BlockSpec(memory_space=pl.ANY),
            pl.BlockSpec((_RI, 16, 128),
                         lambda h, k: (h * (_Q // _RI) + k, 0, 0)),
        ],
        out_specs=pl.BlockSpec(
            (1, 1, _RI, _Q), lambda h, k: (0, h, k, 0)),
        out_shape=jax.ShapeDtypeStruct((1, _NHEAD, _Q, _Q), jnp.float32),
        input_output_aliases={0: 0},
    )(o1, lin3)


def kernel(q_len, k_len, bias_emb):
    if k_len is None:
        k_len = q_len
    t = jnp.arange(_LPAD, dtype=jnp.int32)
    d = t - 2047
    rb = jnp.where(d > 0, 16, 0).astype(jnp.int32)
    a = jnp.abs(d)
    rp_safe = jnp.maximum(a, 1)
    large = 8 + (
        jnp.log(rp_safe.astype(jnp.float32) / 8) / math.log(16.0) * 8
    ).astype(jnp.int32)
    large = jnp.minimum(large, 15)
    bucket = (rb + jnp.where(a < 8, a, large)).reshape(1, _LPAD)
    embT = bias_emb.T
    lineL = _tc_linel(bucket, embT)
    lin = _sc_call(lineL.reshape(_NHEAD * 8 * _LROW))
    l16 = _tc_l16(bucket, embT)
    o1 = _tc_direct(l16)
    lin3 = lin.reshape(_HS * _Q, 16, 128)
    return _tc_finish(o1, lin3)


# speedup vs baseline: 3.1373x; 1.0164x over previous
import math

import jax
import jax.numpy as jnp
from jax import lax
from jax.experimental import pallas as pl
from jax.experimental.pallas import tpu as pltpu
from jax.experimental.pallas import tpu_sc as plsc

_Q = 2048
_NHEAD = 16
_NBUCKET = 32
_HS = 4
_LPAD = 4224
_LROW = 4096
_LROW2 = 3968
_RI = 1024
_RID = 2048


def _line_t(bucket_ref, embT_ref):
    bucket = jnp.broadcast_to(bucket_ref[...], (_NBUCKET, _LPAD))
    ids = lax.broadcasted_iota(jnp.int32, (_NBUCKET, _LPAD), 0)
    oh = (bucket == ids).astype(jnp.float32)
    return jnp.dot(embT_ref[...], oh, preferred_element_type=jnp.float32,
                   precision=lax.Precision.HIGHEST)


def _tc_linel_body(bucket_ref, embT_ref, lineL_ref):
    lineT = _line_t(bucket_ref, embT_ref)
    for r in range(8):
        lineL_ref[:, r, :] = lax.slice(lineT, (0, 7 - r), (_HS, 7 - r + _LROW))


def _tc_linel(bucket, embT):
    return pl.pallas_call(
        _tc_linel_body,
        out_shape=jax.ShapeDtypeStruct((_HS, 8, _LROW), jnp.float32),
    )(bucket, embT)


def _tc_l16_body(bucket_ref, embT_ref, l16_ref):
    lineT = _line_t(bucket_ref, embT_ref)
    for v in range(16):
        for r in range(8):
            s = 8 * v + 7 - r
            l16_ref[:, v, r, :] = lax.slice(lineT, (_HS, s), (_NHEAD, s + _LROW2))


def _tc_l16(bucket, embT):
    return pl.pallas_call(
        _tc_l16_body,
        out_shape=jax.ShapeDtypeStruct((_NHEAD - _HS, 16, 8, _LROW2), jnp.float32),
    )(bucket, embT)


def _sc_body(lineL_hbm, out_hbm, lines_v, sem):
    nc = 2
    wid = lax.axis_index("s") * nc + lax.axis_index("c")
    wph = 32 // _HS
    h = wid // wph
    part = wid % wph
    rows = _Q // wph

    lsz = 8 * _LROW
    pltpu.sync_copy(lineL_hbm.at[pl.ds(pl.multiple_of(h * lsz, 8), lsz)], lines_v)

    base_i = part * rows

    def group8(g, carry):
        i0 = base_i + g * 8
        base = 2040 - i0
        descs = []
        for k in range(8):
            src = pl.multiple_of(k * _LROW + base, 8)
            d = pltpu.make_async_copy(
                lines_v.at[pl.ds(src, _Q)],
                out_hbm.at[pl.ds(pl.multiple_of((h * _Q + i0 + k) * _Q, 8), _Q)],
                sem,
            )
            d.start()
            descs.append(d)
        for d in descs:
            d.wait()
        return carry

    lax.fori_loop(0, rows // 8, group8, 0)


def _sc_call(lineL):
    mesh = plsc.VectorSubcoreMesh(core_axis_name="c", subcore_axis_name="s")
    return pl.kernel(
        _sc_body,
        out_type=jax.ShapeDtypeStruct((_HS * _Q * _Q,), jnp.float32),
        mesh=mesh,
        scratch_types=[
            pltpu.VMEM((8 * _LROW,), jnp.float32),
            pltpu.SemaphoreType.DMA,
        ],
    )(lineL)


def _tc_direct_body(l16_ref, out_ref):
    k = pl.program_id(1)
    for j in range(_RID // 128):
        off = pl.multiple_of(128 * (15 - (_RID // 128) * k - j), 128)
        for qq in range(16):
            q = 16 * j + qq
            out_ref[0, 0, 8 * q:8 * q + 8, :] = \
                l16_ref[0, 15 - qq, :, pl.ds(off, _Q)]


def _tc_direct(l16):
    return pl.pallas_call(
        _tc_direct_body,
        grid=(_NHEAD - _HS, _Q // _RID),
        in_specs=[pl.BlockSpec(
            (1, 16, 8, _LROW2), lambda h, k: (h, 0, 0, 0))],
        out_specs=pl.BlockSpec(
            (1, 1, _RID, _Q), lambda h, k: (0, _HS + h, k, 0)),
        out_shape=jax.ShapeDtypeStruct((1, _NHEAD, _Q, _Q), jnp.float32),
    )(l16)


def _tc_finish_body(o1_ref, lin_ref, out_ref):
    out_ref[0, 0] = lin_ref[...].reshape(_RI, _Q)


def _tc_finish(o1, lin3):
    return pl.pallas_call(
        _tc_finish_body,
        grid=(_HS, _Q // _RI),
        in_specs=[
            pl.BlockSpec(memory_space=pl.ANY),
            pl.BlockSpec((_RI, 16, 128),
                         lambda h, k: (h * (_Q // _RI) + k, 0, 0)),
        ],
        out_specs=pl.BlockSpec(
            (1, 1, _RI, _Q), lambda h, k: (0, h, k, 0)),
        out_shape=jax.ShapeDtypeStruct((1, _NHEAD, _Q, _Q), jnp.float32),
        input_output_aliases={0: 0},
    )(o1, lin3)


def kernel(q_len, k_len, bias_emb):
    if k_len is None:
        k_len = q_len
    t = jnp.arange(_LPAD, dtype=jnp.int32)
    d = t - 2047
    rb = jnp.where(d > 0, 16, 0).astype(jnp.int32)
    a = jnp.abs(d)
    rp_safe = jnp.maximum(a, 1)
    large = 8 + (
        jnp.log(rp_safe.astype(jnp.float32) / 8) / math.log(16.0) * 8
    ).astype(jnp.int32)
    large = jnp.minimum(large, 15)
    bucket = (rb + jnp.where(a < 8, a, large)).reshape(1, _LPAD)
    embT = bias_emb.T
    lineL = _tc_linel(bucket, embT)
    lin = _sc_call(lineL.reshape(_HS * 8 * _LROW))
    l16 = _tc_l16(bucket, embT)
    o1 = _tc_direct(l16)
    lin3 = lin.reshape(_HS * _Q, 16, 128)
    return _tc_finish(o1, lin3)
